# Initial kernel scaffold; baseline (speedup 1.0000x reference)
import jax
import jax.numpy as jnp
from jax.experimental import pallas as pl


def kernel(x, edge_index, W0, b0, g0, be0, rm0, rv0, W1, b1, g1, be1, rm1, rv1, W2, b2, g2, be2, rm2, rv2):
    raise NotImplementedError("write your pallas kernel here")


# speedup vs baseline: 8.0665x; 8.0665x over previous
import functools

import jax
import jax.numpy as jnp
from jax import lax
from jax.experimental import pallas as pl
from jax.experimental.pallas import tpu as pltpu
from jax.experimental.pallas import tpu_sc as plsc

N = 10000
NP = 10240
D = 128
H = 128
EPS = 1e-5

NC = 2
NS = 16
CH = 128
NROW = 2560
EPAD = NROW * CH
RT = NP // NS

_F32 = jnp.float32


@functools.lru_cache(maxsize=1)
def _sc_mesh():
    return plsc.VectorSubcoreMesh(core_axis_name="c", subcore_axis_name="s",
                                  num_cores=NC, num_subcores=NS)


def _deg_body(dst_hbm, zeros_hbm, ones_hbm, deg_out, deg_sh, idx_v, ones_v):
    c = lax.axis_index("c")
    s = lax.axis_index("s")
    nch = NROW // (NC * NS)
    base = (c * NS + s) * nch
    pltpu.sync_copy(zeros_hbm.at[pl.ds(s * RT, RT)],
                    deg_sh.at[pl.ds(s * RT, RT)])
    pltpu.sync_copy(ones_hbm, ones_v)
    pltpu.sync_copy(dst_hbm.at[pl.ds(base, nch)], idx_v)
    plsc.subcore_barrier()

    @pl.loop(0, nch)
    def _(j):
        pltpu.sync_copy(ones_v, deg_sh.at[idx_v.at[j]], add=True)

    plsc.subcore_barrier()
    pltpu.sync_copy(deg_sh.at[pl.ds(s * RT, RT)],
                    deg_out.at[c, pl.ds(s * RT, RT)])


def _deg_call(dstr, zeros8, ones8):
    nch = NROW // (NC * NS)
    return pl.kernel(
        _deg_body,
        out_type=jax.ShapeDtypeStruct((NC, NP, 8), _F32),
        mesh=_sc_mesh(),
        scratch_types=[
            pltpu.VMEM_SHARED((NP, 8), _F32),
            pltpu.VMEM((nch, CH), jnp.int32),
            pltpu.VMEM((CH, 8), _F32),
        ],
    )(dstr, zeros8, ones8)


def _agg_body(u_hbm, src_hbm, dst_hbm, p_out, acc_sh, idx_s, idx_d, rows_v,
              sem):
    c = lax.axis_index("c")
    s = lax.axis_index("s")
    nch = NROW // (NC * NS)
    base = (c * NS + s) * nch
    pltpu.sync_copy(u_hbm.at[pl.ds(s * RT, RT)],
                    acc_sh.at[pl.ds(s * RT, RT)])
    pltpu.sync_copy(src_hbm.at[pl.ds(base, nch)], idx_s)
    pltpu.sync_copy(dst_hbm.at[pl.ds(base, nch)], idx_d)
    plsc.subcore_barrier()

    @pl.loop(0, nch)
    def _(j):
        pltpu.async_copy(u_hbm.at[idx_s.at[j]], rows_v, sem).wait()
        pltpu.sync_copy(rows_v, acc_sh.at[idx_d.at[j]], add=True)

    plsc.subcore_barrier()
    pltpu.sync_copy(acc_sh.at[pl.ds(s * RT, RT)],
                    p_out.at[c, pl.ds(s * RT, RT)])


def _agg_call(u, srcr, dstr):
    nch = NROW // (NC * NS)
    return pl.kernel(
        _agg_body,
        out_type=jax.ShapeDtypeStruct((NC, NP, H), _F32),
        mesh=_sc_mesh(),
        scratch_types=[
            pltpu.VMEM_SHARED((NP, H), _F32),
            pltpu.VMEM((nch, CH), jnp.int32),
            pltpu.VMEM((nch, CH), jnp.int32),
            pltpu.VMEM((CH, H), _F32),
            pltpu.SemaphoreType.DMA,
        ],
    )(u, srcr, dstr)


BR = 1024
NB = NP // BR


def _dinv_of(degp_blk):
    deg = degp_blk[0, :, 0:1] + degp_blk[1, :, 0:1] + 1.0
    return lax.rsqrt(deg)


def _pre_body(x_ref, w_ref, degp_ref, u_ref):
    dinv = _dinv_of(degp_ref[...])
    u_ref[...] = jnp.dot(x_ref[...], w_ref[...],
                         preferred_element_type=_F32) * dinv


def _pre_call(x, W, degp):
    return pl.pallas_call(
        _pre_body,
        grid=(NB,),
        in_specs=[
            pl.BlockSpec((BR, D), lambda r: (r, 0)),
            pl.BlockSpec((D, H), lambda r: (0, 0)),
            pl.BlockSpec((NC, BR, 8), lambda r: (0, r, 0)),
        ],
        out_specs=pl.BlockSpec((BR, H), lambda r: (r, 0)),
        out_shape=jax.ShapeDtypeStruct((NP, H), _F32),
    )(x, W, degp)


def _bn_elu(agg, dinv, b, g, be, rm, rv):
    z = agg * dinv + b
    z = (z - rm) * lax.rsqrt(rv + EPS) * g + be
    return jnp.where(z > 0, z, jnp.exp(jnp.minimum(z, 0.0)) - 1.0)


def _mid_body(p_ref, u_ref, degp_ref, b_ref, g_ref, be_ref, rm_ref, rv_ref,
              w_ref, o_ref):
    dinv = _dinv_of(degp_ref[...])
    agg = p_ref[0] + p_ref[1] - u_ref[...]
    y = _bn_elu(agg, dinv, b_ref[...], g_ref[...], be_ref[...], rm_ref[...],
                rv_ref[...])
    o_ref[...] = jnp.dot(y, w_ref[...], preferred_element_type=_F32) * dinv


def _mid_call(p, u, degp, b, g, be, rm, rv, W):
    vec = pl.BlockSpec((1, H), lambda r: (0, 0))
    return pl.pallas_call(
        _mid_body,
        grid=(NB,),
        in_specs=[
            pl.BlockSpec((NC, BR, H), lambda r: (0, r, 0)),
            pl.BlockSpec((BR, H), lambda r: (r, 0)),
            pl.BlockSpec((NC, BR, 8), lambda r: (0, r, 0)),
            vec, vec, vec, vec, vec,
            pl.BlockSpec((D, H), lambda r: (0, 0)),
        ],
        out_specs=pl.BlockSpec((BR, H), lambda r: (r, 0)),
        out_shape=jax.ShapeDtypeStruct((NP, H), _F32),
    )(p, u, degp, b, g, be, rm, rv, W)


def _final_body(p_ref, u_ref, degp_ref, b_ref, g_ref, be_ref, rm_ref, rv_ref,
                out_ref):
    dinv = _dinv_of(degp_ref[...])
    agg = p_ref[0] + p_ref[1] - u_ref[...]
    out_ref[...] = _bn_elu(agg, dinv, b_ref[...], g_ref[...], be_ref[...],
                           rm_ref[...], rv_ref[...])


def _final_call(p, u, degp, b, g, be, rm, rv):
    vec = pl.BlockSpec((1, H), lambda r: (0, 0))
    return pl.pallas_call(
        _final_body,
        grid=(NB,),
        in_specs=[
            pl.BlockSpec((NC, BR, H), lambda r: (0, r, 0)),
            pl.BlockSpec((BR, H), lambda r: (r, 0)),
            pl.BlockSpec((NC, BR, 8), lambda r: (0, r, 0)),
            vec, vec, vec, vec, vec,
        ],
        out_specs=pl.BlockSpec((BR, H), lambda r: (r, 0)),
        out_shape=jax.ShapeDtypeStruct((N, H), _F32),
    )(p, u, degp, b, g, be, rm, rv)


def kernel(x, edge_index, W0, b0, g0, be0, rm0, rv0, W1, b1, g1, be1, rm1,
           rv1, W2, b2, g2, be2, rm2, rv2):
    E = edge_index.shape[1]
    pad = EPAD - E
    src = edge_index[0]
    dst = edge_index[1]
    srcr = jnp.concatenate([src, jnp.zeros((pad,), jnp.int32)]).reshape(
        NROW, CH)
    dstr = jnp.concatenate([dst, jnp.full((pad,), N, jnp.int32)]).reshape(
        NROW, CH)
    zeros8 = jnp.zeros((NP, 8), _F32)
    ones8 = jnp.ones((CH, 8), _F32)

    degp = _deg_call(dstr, zeros8, ones8)

    r2 = lambda v: v.reshape(1, H)
    u = _pre_call(x, W0, degp)
    p = _agg_call(u, srcr, dstr)
    u = _mid_call(p, u, degp, r2(b0), r2(g0), r2(be0), r2(rm0), r2(rv0), W1)
    p = _agg_call(u, srcr, dstr)
    u = _mid_call(p, u, degp, r2(b1), r2(g1), r2(be1), r2(rm1), r2(rv1), W2)
    p = _agg_call(u, srcr, dstr)
    return _final_call(p, u, degp, r2(b2), r2(g2), r2(be2), r2(rm2), r2(rv2))


# speedup vs baseline: 8.1791x; 1.0140x over previous
import functools

import jax
import jax.numpy as jnp
from jax import lax
from jax.experimental import pallas as pl
from jax.experimental.pallas import tpu as pltpu
from jax.experimental.pallas import tpu_sc as plsc

N = 10000
NP = 10240
D = 128
H = 128
EPS = 1e-5

NC = 2
NS = 16
CH = 128
NROW = 2560
EPAD = NROW * CH
RT = NP // NS

_F32 = jnp.float32


@functools.lru_cache(maxsize=1)
def _sc_mesh():
    return plsc.VectorSubcoreMesh(core_axis_name="c", subcore_axis_name="s",
                                  num_cores=NC, num_subcores=NS)


def _deg_body(dst_hbm, zeros_hbm, ones_hbm, deg_out, deg_sh, idx_v, ones_v):
    c = lax.axis_index("c")
    s = lax.axis_index("s")
    nch = NROW // (NC * NS)
    base = (c * NS + s) * nch
    pltpu.sync_copy(zeros_hbm.at[pl.ds(s * RT, RT)],
                    deg_sh.at[pl.ds(s * RT, RT)])
    pltpu.sync_copy(ones_hbm, ones_v)
    pltpu.sync_copy(dst_hbm.at[pl.ds(base, nch)], idx_v)
    plsc.subcore_barrier()

    @pl.loop(0, nch)
    def _(j):
        pltpu.sync_copy(ones_v, deg_sh.at[idx_v.at[j]], add=True)

    plsc.subcore_barrier()
    pltpu.sync_copy(deg_sh.at[pl.ds(s * RT, RT)],
                    deg_out.at[c, pl.ds(s * RT, RT)])


def _deg_call(dstr, zeros8, ones8):
    nch = NROW // (NC * NS)
    return pl.kernel(
        _deg_body,
        out_type=jax.ShapeDtypeStruct((NC, NP, 8), _F32),
        mesh=_sc_mesh(),
        scratch_types=[
            pltpu.VMEM_SHARED((NP, 8), _F32),
            pltpu.VMEM((nch, CH), jnp.int32),
            pltpu.VMEM((CH, 8), _F32),
        ],
    )(dstr, zeros8, ones8)


KB = 2
IG = 16


def _agg_body(u_hbm, src_hbm, dst_hbm, p_out, acc_sh, idx_s, idx_d, rows_v,
              g0, g1):
    c = lax.axis_index("c")
    s = lax.axis_index("s")
    gsems = (g0, g1)
    nch = NROW // (NC * NS)
    base = (c * NS + s) * nch
    pltpu.sync_copy(u_hbm.at[pl.ds(s * RT, RT)],
                    acc_sh.at[pl.ds(s * RT, RT)])
    plsc.subcore_barrier()

    @pl.loop(0, nch // IG)
    def _(gi):
        gb = base + gi * IG
        pltpu.sync_copy(src_hbm.at[pl.ds(gb, IG)], idx_s)
        pltpu.sync_copy(dst_hbm.at[pl.ds(gb, IG)], idx_d)

        @pl.loop(0, IG // KB)
        def _(ii):
            jb = ii * KB
            descs = []
            for b in range(KB):
                descs.append(pltpu.async_copy(u_hbm.at[idx_s.at[jb + b]],
                                              rows_v.at[b], gsems[b]))
            for b in range(KB):
                descs[b].wait()
                pltpu.sync_copy(rows_v.at[b], acc_sh.at[idx_d.at[jb + b]],
                                add=True)

    plsc.subcore_barrier()
    pltpu.sync_copy(acc_sh.at[pl.ds(s * RT, RT)],
                    p_out.at[c, pl.ds(s * RT, RT)])


def _agg_call(u, srcr, dstr):
    return pl.kernel(
        _agg_body,
        out_type=jax.ShapeDtypeStruct((NC, NP, H), _F32),
        mesh=_sc_mesh(),
        scratch_types=[
            pltpu.VMEM_SHARED((NP, H), _F32),
            pltpu.VMEM((IG, CH), jnp.int32),
            pltpu.VMEM((IG, CH), jnp.int32),
            pltpu.VMEM((KB, CH, H), _F32),
            pltpu.SemaphoreType.DMA,
            pltpu.SemaphoreType.DMA,
        ],
    )(u, srcr, dstr)


BR = 1024
NB = NP // BR


def _dinv_of(degp_blk):
    deg = degp_blk[0, :, 0:1] + degp_blk[1, :, 0:1] + 1.0
    return lax.rsqrt(deg)


def _pre_body(x_ref, w_ref, degp_ref, u_ref):
    dinv = _dinv_of(degp_ref[...])
    u_ref[...] = jnp.dot(x_ref[...], w_ref[...],
                         preferred_element_type=_F32) * dinv


def _pre_call(x, W, degp):
    return pl.pallas_call(
        _pre_body,
        grid=(NB,),
        in_specs=[
            pl.BlockSpec((BR, D), lambda r: (r, 0)),
            pl.BlockSpec((D, H), lambda r: (0, 0)),
            pl.BlockSpec((NC, BR, 8), lambda r: (0, r, 0)),
        ],
        out_specs=pl.BlockSpec((BR, H), lambda r: (r, 0)),
        out_shape=jax.ShapeDtypeStruct((NP, H), _F32),
    )(x, W, degp)


def _bn_elu(agg, dinv, b, g, be, rm, rv):
    z = agg * dinv + b
    z = (z - rm) * lax.rsqrt(rv + EPS) * g + be
    return jnp.where(z > 0, z, jnp.exp(jnp.minimum(z, 0.0)) - 1.0)


def _mid_body(p_ref, u_ref, degp_ref, b_ref, g_ref, be_ref, rm_ref, rv_ref,
              w_ref, o_ref):
    dinv = _dinv_of(degp_ref[...])
    agg = p_ref[0] + p_ref[1] - u_ref[...]
    y = _bn_elu(agg, dinv, b_ref[...], g_ref[...], be_ref[...], rm_ref[...],
                rv_ref[...])
    o_ref[...] = jnp.dot(y, w_ref[...], preferred_element_type=_F32) * dinv


def _mid_call(p, u, degp, b, g, be, rm, rv, W):
    vec = pl.BlockSpec((1, H), lambda r: (0, 0))
    return pl.pallas_call(
        _mid_body,
        grid=(NB,),
        in_specs=[
            pl.BlockSpec((NC, BR, H), lambda r: (0, r, 0)),
            pl.BlockSpec((BR, H), lambda r: (r, 0)),
            pl.BlockSpec((NC, BR, 8), lambda r: (0, r, 0)),
            vec, vec, vec, vec, vec,
            pl.BlockSpec((D, H), lambda r: (0, 0)),
        ],
        out_specs=pl.BlockSpec((BR, H), lambda r: (r, 0)),
        out_shape=jax.ShapeDtypeStruct((NP, H), _F32),
    )(p, u, degp, b, g, be, rm, rv, W)


def _final_body(p_ref, u_ref, degp_ref, b_ref, g_ref, be_ref, rm_ref, rv_ref,
                out_ref):
    dinv = _dinv_of(degp_ref[...])
    agg = p_ref[0] + p_ref[1] - u_ref[...]
    out_ref[...] = _bn_elu(agg, dinv, b_ref[...], g_ref[...], be_ref[...],
                           rm_ref[...], rv_ref[...])


def _final_call(p, u, degp, b, g, be, rm, rv):
    vec = pl.BlockSpec((1, H), lambda r: (0, 0))
    return pl.pallas_call(
        _final_body,
        grid=(NB,),
        in_specs=[
            pl.BlockSpec((NC, BR, H), lambda r: (0, r, 0)),
            pl.BlockSpec((BR, H), lambda r: (r, 0)),
            pl.BlockSpec((NC, BR, 8), lambda r: (0, r, 0)),
            vec, vec, vec, vec, vec,
        ],
        out_specs=pl.BlockSpec((BR, H), lambda r: (r, 0)),
        out_shape=jax.ShapeDtypeStruct((N, H), _F32),
    )(p, u, degp, b, g, be, rm, rv)


def kernel(x, edge_index, W0, b0, g0, be0, rm0, rv0, W1, b1, g1, be1, rm1,
           rv1, W2, b2, g2, be2, rm2, rv2):
    E = edge_index.shape[1]
    pad = EPAD - E
    src = edge_index[0]
    dst = edge_index[1]
    srcr = jnp.concatenate([src, jnp.zeros((pad,), jnp.int32)]).reshape(
        NROW, CH)
    dstr = jnp.concatenate([dst, jnp.full((pad,), N, jnp.int32)]).reshape(
        NROW, CH)
    zeros8 = jnp.zeros((NP, 8), _F32)
    ones8 = jnp.ones((CH, 8), _F32)

    degp = _deg_call(dstr, zeros8, ones8)

    r2 = lambda v: v.reshape(1, H)
    u = _pre_call(x, W0, degp)
    p = _agg_call(u, srcr, dstr)
    u = _mid_call(p, u, degp, r2(b0), r2(g0), r2(be0), r2(rm0), r2(rv0), W1)
    p = _agg_call(u, srcr, dstr)
    u = _mid_call(p, u, degp, r2(b1), r2(g1), r2(be1), r2(rm1), r2(rv1), W2)
    p = _agg_call(u, srcr, dstr)
    return _final_call(p, u, degp, r2(b2), r2(g2), r2(be2), r2(rm2), r2(rv2))
